# Initial kernel scaffold; baseline (speedup 1.0000x reference)
import jax
import jax.numpy as jnp
from jax.experimental import pallas as pl


def kernel(x, W1, b1, gamma, beta, W2, b2, W3, b3, W4, b4, emb):
    raise NotImplementedError("write your pallas kernel here")


# speedup vs baseline: 1.7550x; 1.7550x over previous
import functools

import jax
import jax.numpy as jnp
from jax.experimental import pallas as pl

_OBS = 768
_HID = 1024
_LAT = 256
_VOCAB = 1024
_HQ = 2
_BATCH = 16384
_COMMIT = 0.5
_BM = 512


def _body(x_ref, W1_ref, b1_ref, gamma_ref, beta_ref, W2_ref, b2_ref,
          W3_ref, b3_ref, W4_ref, b4_ref, emb_ref, out_ref):
    xb = x_ref[...]
    h = jnp.dot(xb, W1_ref[...], preferred_element_type=jnp.float32) + b1_ref[...]
    mu = jnp.mean(h, axis=1, keepdims=True)
    hc = h - mu
    var = jnp.mean(hc * hc, axis=1, keepdims=True)
    h = hc * jax.lax.rsqrt(var + 1e-5) * gamma_ref[...] + beta_ref[...]
    h = jnp.maximum(h, 0.0)
    latent = jnp.dot(h, W2_ref[...], preferred_element_type=jnp.float32) + b2_ref[...]

    curr = latent
    code_sum = jnp.zeros_like(latent)
    loss = jnp.float32(0.0)
    for l in range(_HQ):
        E = emb_ref[l]
        cc = jnp.sum(curr * curr, axis=1, keepdims=True)
        ee = jnp.sum(E * E, axis=1)[None, :]
        ce = jax.lax.dot_general(curr, E, (((1,), (1,)), ((), ())),
                                 preferred_element_type=jnp.float32)
        d2 = cc + ee - 2.0 * ce
        m = jnp.min(d2, axis=1, keepdims=True)
        iota = jax.lax.broadcasted_iota(jnp.int32, d2.shape, 1)
        idx = jnp.min(jnp.where(d2 == m, iota, _VOCAB), axis=1, keepdims=True)
        onehot = (iota == idx).astype(jnp.float32)
        q = jnp.dot(onehot, E, preferred_element_type=jnp.float32)
        r = curr - q
        loss += (1.0 + _COMMIT) * jnp.sum(r * r) / (_BATCH * _LAT)
        code_sum = code_sum + q
        curr = r

    h2 = jnp.maximum(
        jnp.dot(code_sum, W3_ref[...], preferred_element_type=jnp.float32) + b3_ref[...], 0.0)
    recon = jnp.dot(h2, W4_ref[...], preferred_element_type=jnp.float32) + b4_ref[...]
    e = recon - xb
    loss += 0.5 * jnp.sum(e * e) / (_BATCH * _OBS)

    loss_arr = jnp.reshape(loss, (1, 1))

    @pl.when(pl.program_id(0) == 0)
    def _init():
        out_ref[...] = loss_arr

    @pl.when(pl.program_id(0) != 0)
    def _acc():
        out_ref[...] += loss_arr


@functools.partial(jax.jit, static_argnames=())
def kernel(x, W1, b1, gamma, beta, W2, b2, W3, b3, W4, b4, emb):
    grid = _BATCH // _BM
    full = lambda shape: pl.BlockSpec(shape, lambda i: (0,) * len(shape))
    out = pl.pallas_call(
        _body,
        grid=(grid,),
        in_specs=[
            pl.BlockSpec((_BM, _OBS), lambda i: (i, 0)),
            full((_OBS, _HID)),
            full((1, _HID)),
            full((1, _HID)),
            full((1, _HID)),
            full((_HID, _LAT)),
            full((1, _LAT)),
            full((_LAT, _HID)),
            full((1, _HID)),
            full((_HID, _OBS)),
            full((1, _OBS)),
            full((_HQ, _VOCAB, _LAT)),
        ],
        out_specs=pl.BlockSpec((1, 1), lambda i: (0, 0)),
        out_shape=jax.ShapeDtypeStruct((1, 1), jnp.float32),
    )(x, W1, b1.reshape(1, -1), gamma.reshape(1, -1), beta.reshape(1, -1),
      W2, b2.reshape(1, -1), W3, b3.reshape(1, -1), W4, b4.reshape(1, -1), emb)
    return out[0, 0]


# speedup vs baseline: 1.7643x; 1.0053x over previous
import functools

import jax
import jax.numpy as jnp
from jax.experimental import pallas as pl
from jax.experimental.pallas import tpu as pltpu

_OBS = 768
_HID = 1024
_LAT = 256
_VOCAB = 1024
_HQ = 2
_BATCH = 16384
_COMMIT = 0.5
_BM = 512


def _body(x_ref, W1_ref, b1_ref, gamma_ref, beta_ref, W2_ref, b2_ref,
          W3_ref, b3_ref, W4_ref, b4_ref, emb_ref, out_ref, ee0_ref, ee1_ref):
    ee_refs = (ee0_ref, ee1_ref)

    @pl.when(pl.program_id(0) == 0)
    def _norms():
        for l in range(_HQ):
            E = emb_ref[l]
            ee_refs[l][...] = 0.5 * jnp.sum(E * E, axis=1)[None, :]

    xb = x_ref[...]
    h = jnp.dot(xb, W1_ref[...], preferred_element_type=jnp.float32) + b1_ref[...]
    mu = jnp.mean(h, axis=1, keepdims=True)
    hc = h - mu
    var = jnp.mean(hc * hc, axis=1, keepdims=True)
    h = hc * jax.lax.rsqrt(var + 1e-5) * gamma_ref[...] + beta_ref[...]
    h = jnp.maximum(h, 0.0)
    latent = jnp.dot(h, W2_ref[...], preferred_element_type=jnp.float32) + b2_ref[...]

    curr = latent
    code_sum = jnp.zeros_like(latent)
    loss = jnp.float32(0.0)
    for l in range(_HQ):
        E = emb_ref[l]
        ce = jax.lax.dot_general(curr, E, (((1,), (1,)), ((), ())),
                                 preferred_element_type=jnp.float32)
        score = ce - ee_refs[l][...]
        m = jnp.max(score, axis=1, keepdims=True)
        iota = jax.lax.broadcasted_iota(jnp.int32, score.shape, 1)
        idx = jnp.min(jnp.where(score == m, iota, _VOCAB), axis=1, keepdims=True)
        onehot = (iota == idx).astype(jnp.float32)
        q = jnp.dot(onehot, E, preferred_element_type=jnp.float32)
        r = curr - q
        loss += (1.0 + _COMMIT) * jnp.sum(r * r) / (_BATCH * _LAT)
        code_sum = code_sum + q
        curr = r

    h2 = jnp.maximum(
        jnp.dot(code_sum, W3_ref[...], preferred_element_type=jnp.float32) + b3_ref[...], 0.0)
    recon = jnp.dot(h2, W4_ref[...], preferred_element_type=jnp.float32) + b4_ref[...]
    e = recon - xb
    loss += 0.5 * jnp.sum(e * e) / (_BATCH * _OBS)

    loss_arr = jnp.reshape(loss, (1, 1))

    @pl.when(pl.program_id(0) == 0)
    def _init():
        out_ref[...] = loss_arr

    @pl.when(pl.program_id(0) != 0)
    def _acc():
        out_ref[...] += loss_arr


@functools.partial(jax.jit, static_argnames=())
def kernel(x, W1, b1, gamma, beta, W2, b2, W3, b3, W4, b4, emb):
    grid = _BATCH // _BM
    full = lambda shape: pl.BlockSpec(shape, lambda i: (0,) * len(shape))
    out = pl.pallas_call(
        _body,
        grid=(grid,),
        in_specs=[
            pl.BlockSpec((_BM, _OBS), lambda i: (i, 0)),
            full((_OBS, _HID)),
            full((1, _HID)),
            full((1, _HID)),
            full((1, _HID)),
            full((_HID, _LAT)),
            full((1, _LAT)),
            full((_LAT, _HID)),
            full((1, _HID)),
            full((_HID, _OBS)),
            full((1, _OBS)),
            full((_HQ, _VOCAB, _LAT)),
        ],
        out_specs=pl.BlockSpec((1, 1), lambda i: (0, 0)),
        out_shape=jax.ShapeDtypeStruct((1, 1), jnp.float32),
        scratch_shapes=[pltpu.VMEM((1, _VOCAB), jnp.float32),
                        pltpu.VMEM((1, _VOCAB), jnp.float32)],
    )(x, W1, b1.reshape(1, -1), gamma.reshape(1, -1), beta.reshape(1, -1),
      W2, b2.reshape(1, -1), W3, b3.reshape(1, -1), W4, b4.reshape(1, -1), emb)
    return out[0, 0]


# speedup vs baseline: 1.7683x; 1.0023x over previous
import functools

import jax
import jax.numpy as jnp
from jax.experimental import pallas as pl
from jax.experimental.pallas import tpu as pltpu

_OBS = 768
_HID = 1024
_LAT = 256
_VOCAB = 1024
_HQ = 2
_BATCH = 16384
_COMMIT = 0.5
_BM = 512


def _body(x_ref, W1_ref, b1_ref, gamma_ref, beta_ref, W2_ref, b2_ref,
          W3_ref, b3_ref, W4_ref, b4_ref, emb_ref, out_ref):
    xb = x_ref[...]
    h = jnp.dot(xb, W1_ref[...], preferred_element_type=jnp.float32) + b1_ref[...]
    mu = jnp.mean(h, axis=1, keepdims=True)
    hc = h - mu
    var = jnp.mean(hc * hc, axis=1, keepdims=True)
    h = hc * jax.lax.rsqrt(var + 1e-5) * gamma_ref[...] + beta_ref[...]
    h = jnp.maximum(h, 0.0)
    latent = jnp.dot(h, W2_ref[...], preferred_element_type=jnp.float32) + b2_ref[...]

    curr = latent
    code_sum = jnp.zeros_like(latent)
    loss = jnp.float32(0.0)
    for l in range(_HQ):
        E = emb_ref[l]
        half_ee = 0.5 * jnp.sum(E * E, axis=1)[None, :]
        ce = jax.lax.dot_general(curr, E, (((1,), (1,)), ((), ())),
                                 preferred_element_type=jnp.float32)
        score = ce - half_ee
        m = jnp.max(score, axis=1, keepdims=True)
        iota = jax.lax.broadcasted_iota(jnp.int32, score.shape, 1)
        idx = jnp.min(jnp.where(score == m, iota, _VOCAB), axis=1, keepdims=True)
        onehot = (iota == idx).astype(jnp.float32)
        q = jnp.dot(onehot, E, preferred_element_type=jnp.float32)
        r = curr - q
        loss += (1.0 + _COMMIT) * jnp.sum(r * r) / (_BATCH * _LAT)
        code_sum = code_sum + q
        curr = r

    h2 = jnp.maximum(
        jnp.dot(code_sum, W3_ref[...], preferred_element_type=jnp.float32) + b3_ref[...], 0.0)
    recon = jnp.dot(h2, W4_ref[...], preferred_element_type=jnp.float32) + b4_ref[...]
    e = recon - xb
    loss += 0.5 * jnp.sum(e * e) / (_BATCH * _OBS)

    out_ref[...] = jnp.reshape(loss, (1, 1, 1))


@functools.partial(jax.jit, static_argnames=())
def kernel(x, W1, b1, gamma, beta, W2, b2, W3, b3, W4, b4, emb):
    grid = _BATCH // _BM
    full = lambda shape: pl.BlockSpec(shape, lambda i: (0,) * len(shape))
    partials = pl.pallas_call(
        _body,
        grid=(grid,),
        in_specs=[
            pl.BlockSpec((_BM, _OBS), lambda i: (i, 0)),
            full((_OBS, _HID)),
            full((1, _HID)),
            full((1, _HID)),
            full((1, _HID)),
            full((_HID, _LAT)),
            full((1, _LAT)),
            full((_LAT, _HID)),
            full((1, _HID)),
            full((_HID, _OBS)),
            full((1, _OBS)),
            full((_HQ, _VOCAB, _LAT)),
        ],
        out_specs=pl.BlockSpec((1, 1, 1), lambda i: (i, 0, 0)),
        out_shape=jax.ShapeDtypeStruct((grid, 1, 1), jnp.float32),
        compiler_params=pltpu.CompilerParams(
            dimension_semantics=("parallel",)),
    )(x, W1, b1.reshape(1, -1), gamma.reshape(1, -1), beta.reshape(1, -1),
      W2, b2.reshape(1, -1), W3, b3.reshape(1, -1), W4, b4.reshape(1, -1), emb)
    return jnp.sum(partials)


# speedup vs baseline: 2.2124x; 1.2512x over previous
import functools

import jax
import jax.numpy as jnp
from jax.experimental import pallas as pl
from jax.experimental.pallas import tpu as pltpu

_OBS = 768
_HID = 1024
_LAT = 256
_VOCAB = 1024
_HQ = 2
_BATCH = 16384
_COMMIT = 0.5
_BM = 1024
_HM = 256
_NH = _BM // _HM


def _bdot(a, b):
    return jnp.dot(a, b, preferred_element_type=jnp.float32)


def _body(x_ref, xb16_ref, W1_ref, b1_ref, gamma_ref, beta_ref, W2_ref, b2_ref,
          W3_ref, b3_ref, W4_ref, b4_ref, emb_ref, embf_ref, out_ref,
          ee0_ref, ee1_ref):
    ee_refs = (ee0_ref, ee1_ref)

    @pl.when(pl.program_id(0) == 0)
    def _norms():
        for l in range(_HQ):
            Ef = embf_ref[l]
            ee_refs[l][...] = 0.5 * jnp.sum(Ef * Ef, axis=1)[None, :]

    S = range(_NH)
    xb = [x_ref[pl.ds(s * _HM, _HM), :] for s in S]
    h = [_bdot(xb16_ref[pl.ds(s * _HM, _HM), :], W1_ref[...]) + b1_ref[...]
         for s in S]
    mu = [jnp.mean(h[s], axis=1, keepdims=True) for s in S]
    hc = [h[s] - mu[s] for s in S]
    var = [jnp.mean(hc[s] * hc[s], axis=1, keepdims=True) for s in S]
    hn = [jnp.maximum(hc[s] * jax.lax.rsqrt(var[s] + 1e-5) * gamma_ref[...]
                      + beta_ref[...], 0.0) for s in S]
    latent = [_bdot(hn[s].astype(jnp.bfloat16), W2_ref[...]) + b2_ref[...]
              for s in S]

    curr = list(latent)
    code_sum = [jnp.zeros_like(latent[s]) for s in S]
    loss = [jnp.float32(0.0) for s in S]
    for l in range(_HQ):
        E16 = emb_ref[l]
        half_ee = ee_refs[l][...]
        ce = [jax.lax.dot_general(curr[s].astype(jnp.bfloat16), E16,
                                  (((1,), (1,)), ((), ())),
                                  preferred_element_type=jnp.float32)
              for s in S]
        score = [ce[s] - half_ee for s in S]
        m = [jnp.max(score[s], axis=1, keepdims=True) for s in S]
        iota = jax.lax.broadcasted_iota(jnp.int32, (_HM, _VOCAB), 1)
        idx = [jnp.min(jnp.where(score[s] == m[s], iota, _VOCAB), axis=1,
                       keepdims=True) for s in S]
        onehot = [(iota == idx[s]).astype(jnp.bfloat16) for s in S]
        q = [_bdot(onehot[s], E16) for s in S]
        r = [curr[s] - q[s] for s in S]
        for s in S:
            loss[s] += (1.0 + _COMMIT) * jnp.sum(r[s] * r[s]) / (_BATCH * _LAT)
            code_sum[s] = code_sum[s] + q[s]
            curr[s] = r[s]

    h2 = [jnp.maximum(_bdot(code_sum[s].astype(jnp.bfloat16), W3_ref[...])
                      + b3_ref[...], 0.0) for s in S]
    recon = [_bdot(h2[s].astype(jnp.bfloat16), W4_ref[...]) + b4_ref[...]
             for s in S]
    e = [recon[s] - xb[s] for s in S]
    total = jnp.float32(0.0)
    for s in S:
        total += loss[s] + 0.5 * jnp.sum(e[s] * e[s]) / (_BATCH * _OBS)

    out_ref[...] = jnp.reshape(total, (1, 1, 1))


@functools.partial(jax.jit, static_argnames=())
def kernel(x, W1, b1, gamma, beta, W2, b2, W3, b3, W4, b4, emb):
    grid = _BATCH // _BM
    full = lambda shape: pl.BlockSpec(shape, lambda i: (0,) * len(shape))
    bf = jnp.bfloat16
    partials = pl.pallas_call(
        _body,
        grid=(grid,),
        in_specs=[
            pl.BlockSpec((_BM, _OBS), lambda i: (i, 0)),
            pl.BlockSpec((_BM, _OBS), lambda i: (i, 0)),
            full((_OBS, _HID)),
            full((1, _HID)),
            full((1, _HID)),
            full((1, _HID)),
            full((_HID, _LAT)),
            full((1, _LAT)),
            full((_LAT, _HID)),
            full((1, _HID)),
            full((_HID, _OBS)),
            full((1, _OBS)),
            full((_HQ, _VOCAB, _LAT)),
            full((_HQ, _VOCAB, _LAT)),
        ],
        out_specs=pl.BlockSpec((1, 1, 1), lambda i: (i, 0, 0)),
        out_shape=jax.ShapeDtypeStruct((grid, 1, 1), jnp.float32),
        scratch_shapes=[pltpu.VMEM((1, _VOCAB), jnp.float32),
                        pltpu.VMEM((1, _VOCAB), jnp.float32)],
    )(x, x.astype(bf), W1.astype(bf), b1.reshape(1, -1), gamma.reshape(1, -1),
      beta.reshape(1, -1), W2.astype(bf), b2.reshape(1, -1), W3.astype(bf),
      b3.reshape(1, -1), W4.astype(bf), b4.reshape(1, -1), emb.astype(bf), emb)
    return jnp.sum(partials)
